# baseline (speedup 1.0000x reference)
import jax
import jax.numpy as jnp
from jax import lax
from jax.experimental import pallas as pl
from jax.experimental.pallas import tpu as pltpu
from jax.experimental.pallas import tpu_sc as plsc

N_NODES = 100000
K = 32
D = 128
B = 16384

NC = 2
NS = 16
NW = NC * NS
BPW = B // NW
C = 2
NCH = BPW // C
LANES = 16
NV = D // LANES
INV_K = 1.0 / K
ISLC = 128


def _gcn_body(x_hbm, adj_hbm, table_hbm, out_hbm,
              x_v, edge_v, self_v, nb0, nb1, nb2, out0, out1, out2,
              sem_e, sem_s, sem_n0, sem_n1, sem_n2, sem_o0, sem_o1, sem_o2):
    wid = lax.axis_index("s") * NC + lax.axis_index("c")
    base = wid * BPW

    pltpu.sync_copy(x_hbm.at[pl.ds(base, BPW)], x_v)

    for j in range(BPW // ISLC):
        sl = pl.ds(j * ISLC, ISLC)
        pltpu.async_copy(adj_hbm.at[x_v.at[sl]], edge_v.at[sl], sem_e)
    for j in range(BPW // ISLC):
        sl = pl.ds(j * ISLC, ISLC)
        pltpu.async_copy(table_hbm.at[x_v.at[sl]], self_v.at[sl], sem_s)
    for j in range(BPW // ISLC):
        sl = pl.ds(j * ISLC, ISLC)
        pltpu.make_async_copy(adj_hbm.at[x_v.at[sl]], edge_v.at[sl], sem_e).wait()

    def fire_nb(g, nb, sem):
        for q in range(C):
            pltpu.async_copy(table_hbm.at[edge_v.at[g * C + q]], nb.at[q], sem)

    def drain_nb(g, nb, sem):
        for q in range(C):
            pltpu.make_async_copy(
                table_hbm.at[edge_v.at[g * C + q]], nb.at[q], sem).wait()

    def fire_out(g, out_v, sem):
        pltpu.async_copy(out_v, out_hbm.at[pl.ds(base + g * C, C)], sem)

    def drain_out(g, out_v, sem):
        pltpu.make_async_copy(
            out_v, out_hbm.at[pl.ds(base + g * C, C)], sem).wait()

    def compute(g, nb, out_v):
        for q in range(C):
            accs = [nb[q, 0, pl.ds(d * LANES, LANES)] for d in range(NV)]
            for k in range(1, K):
                for d in range(NV):
                    accs[d] = accs[d] + nb[q, k, pl.ds(d * LANES, LANES)]
            for d in range(NV):
                dsl = pl.ds(d * LANES, LANES)
                out_v[q, dsl] = accs[d] * INV_K + self_v[g * C + q, dsl]

    fire_nb(0, nb0, sem_n0)
    fire_nb(1, nb1, sem_n1)
    fire_nb(2, nb2, sem_n2)
    for j in range(BPW // ISLC):
        sl = pl.ds(j * ISLC, ISLC)
        pltpu.make_async_copy(table_hbm.at[x_v.at[sl]], self_v.at[sl], sem_s).wait()

    NB = 3
    bufs = ((nb0, sem_n0, out0, sem_o0), (nb1, sem_n1, out1, sem_o1),
            (nb2, sem_n2, out2, sem_o2))

    def step(i, carry):
        for b, (nb, semn, out_v, semo) in enumerate(bufs):
            g = NB * i + b

            @pl.when(g >= NB)
            def _():
                drain_out(g - NB, out_v, semo)

            drain_nb(g, nb, semn)
            compute(g, nb, out_v)
            fire_out(g, out_v, semo)

            @pl.when(g + NB < NCH)
            def _():
                fire_nb(g + NB, nb, semn)

        return carry

    lax.fori_loop(0, NCH // NB, step, 0)
    g_last = (NCH // NB) * NB
    drain_out(g_last - NB, out0, sem_o0)
    drain_nb(g_last, nb0, sem_n0)
    compute(g_last, nb0, out0)
    fire_out(g_last, out0, sem_o0)
    drain_out(g_last - 2, out1, sem_o1)
    drain_out(g_last - 1, out2, sem_o2)
    drain_out(g_last, out0, sem_o0)


def kernel(X, adj, table):
    x = jnp.reshape(X, (B,)).astype(jnp.int32)
    adj32 = adj.astype(jnp.int32)
    f = pl.kernel(
        _gcn_body,
        out_type=jax.ShapeDtypeStruct((B, D), jnp.float32),
        mesh=plsc.VectorSubcoreMesh(core_axis_name="c", subcore_axis_name="s"),
        compiler_params=pltpu.CompilerParams(use_tc_tiling_on_sc=False),
        scratch_types=[
            pltpu.VMEM((BPW,), jnp.int32),
            pltpu.VMEM((BPW, K), jnp.int32),
            pltpu.VMEM((BPW, D), jnp.float32),
            pltpu.VMEM((C, K, D), jnp.float32),
            pltpu.VMEM((C, K, D), jnp.float32),
            pltpu.VMEM((C, K, D), jnp.float32),
            pltpu.VMEM((C, D), jnp.float32),
            pltpu.VMEM((C, D), jnp.float32),
            pltpu.VMEM((C, D), jnp.float32),
            pltpu.SemaphoreType.DMA,
            pltpu.SemaphoreType.DMA,
            pltpu.SemaphoreType.DMA,
            pltpu.SemaphoreType.DMA,
            pltpu.SemaphoreType.DMA,
            pltpu.SemaphoreType.DMA,
            pltpu.SemaphoreType.DMA,
            pltpu.SemaphoreType.DMA,
        ],
    )
    out = f(x, adj32, table)
    return jnp.reshape(out, (B, 1, D))


# speedup vs baseline: 1.3389x; 1.3389x over previous
import jax
import jax.numpy as jnp
from jax import lax
from jax.experimental import pallas as pl
from jax.experimental.pallas import tpu as pltpu
from jax.experimental.pallas import tpu_sc as plsc

N_NODES = 100000
K = 32
D = 128
B = 16384

NC = 2
NS = 16
NW = NC * NS
BPW = B // NW
C = 2
NCH = BPW // C
LANES = 16
NV = D // LANES
INV_K = 1.0 / K
ISLC = 128


def _gcn_body(x_hbm, adj_hbm, table_hbm, out_hbm,
              x_v, edge_v, self_v, nb0, nb1, out0, out1,
              sem_e, sem_s, sem_n0, sem_n1, sem_o0, sem_o1):
    wid = lax.axis_index("s") * NC + lax.axis_index("c")
    base = wid * BPW

    pltpu.sync_copy(x_hbm.at[pl.ds(base, BPW)], x_v)

    for j in range(BPW // ISLC):
        sl = pl.ds(j * ISLC, ISLC)
        pltpu.async_copy(adj_hbm.at[x_v.at[sl]], edge_v.at[sl], sem_e)
    for j in range(BPW // ISLC):
        sl = pl.ds(j * ISLC, ISLC)
        pltpu.async_copy(table_hbm.at[x_v.at[sl]], self_v.at[sl], sem_s)
    for j in range(BPW // ISLC):
        sl = pl.ds(j * ISLC, ISLC)
        pltpu.make_async_copy(adj_hbm.at[x_v.at[sl]], edge_v.at[sl], sem_e).wait()

    def fire_nb(g, nb, sem):
        for q in range(C):
            pltpu.async_copy(table_hbm.at[edge_v.at[g * C + q]], nb.at[q], sem)

    def drain_nb(g, nb, sem):
        for q in range(C):
            pltpu.make_async_copy(
                table_hbm.at[edge_v.at[g * C + q]], nb.at[q], sem).wait()

    def fire_out(g, out_v, sem):
        pltpu.async_copy(out_v, out_hbm.at[pl.ds(base + g * C, C)], sem)

    def drain_out(g, out_v, sem):
        pltpu.make_async_copy(
            out_v, out_hbm.at[pl.ds(base + g * C, C)], sem).wait()

    UNROLL = 4

    def compute(g, nb, out_v):
        for q in range(C):
            def red(k4, accs, q=q):
                new = list(accs)
                for dk in range(UNROLL):
                    row = UNROLL * k4 + dk
                    for d in range(NV):
                        new[d] = new[d] + nb[q, row, pl.ds(d * LANES, LANES)]
                return tuple(new)

            zero = jnp.zeros((LANES,), jnp.float32)
            accs = lax.fori_loop(0, K // UNROLL, red, (zero,) * NV)
            for d in range(NV):
                dsl = pl.ds(d * LANES, LANES)
                out_v[q, dsl] = accs[d] * INV_K + self_v[g * C + q, dsl]

    fire_nb(0, nb0, sem_n0)
    fire_nb(1, nb1, sem_n1)
    for j in range(BPW // ISLC):
        sl = pl.ds(j * ISLC, ISLC)
        pltpu.make_async_copy(table_hbm.at[x_v.at[sl]], self_v.at[sl], sem_s).wait()

    bufs = ((nb0, sem_n0, out0, sem_o0), (nb1, sem_n1, out1, sem_o1))

    def step(i, carry):
        for b, (nb, semn, out_v, semo) in enumerate(bufs):
            g = 2 * i + b

            @pl.when(g >= 2)
            def _():
                drain_out(g - 2, out_v, semo)

            drain_nb(g, nb, semn)
            compute(g, nb, out_v)
            fire_out(g, out_v, semo)

            @pl.when(g + 2 < NCH)
            def _():
                fire_nb(g + 2, nb, semn)

        return carry

    lax.fori_loop(0, NCH // 2, step, 0)
    drain_out(NCH - 2, out0, sem_o0)
    drain_out(NCH - 1, out1, sem_o1)


def kernel(X, adj, table):
    x = jnp.reshape(X, (B,)).astype(jnp.int32)
    adj32 = adj.astype(jnp.int32)
    f = pl.kernel(
        _gcn_body,
        out_type=jax.ShapeDtypeStruct((B, D), jnp.float32),
        mesh=plsc.VectorSubcoreMesh(core_axis_name="c", subcore_axis_name="s"),
        compiler_params=pltpu.CompilerParams(use_tc_tiling_on_sc=False),
        scratch_types=[
            pltpu.VMEM((BPW,), jnp.int32),
            pltpu.VMEM((BPW, K), jnp.int32),
            pltpu.VMEM((BPW, D), jnp.float32),
            pltpu.VMEM((C, K, D), jnp.float32),
            pltpu.VMEM((C, K, D), jnp.float32),
            pltpu.VMEM((C, D), jnp.float32),
            pltpu.VMEM((C, D), jnp.float32),
            pltpu.SemaphoreType.DMA,
            pltpu.SemaphoreType.DMA,
            pltpu.SemaphoreType.DMA,
            pltpu.SemaphoreType.DMA,
            pltpu.SemaphoreType.DMA,
            pltpu.SemaphoreType.DMA,
        ],
    )
    out = f(x, adj32, table)
    return jnp.reshape(out, (B, 1, D))


# speedup vs baseline: 1.6028x; 1.1970x over previous
import jax
import jax.numpy as jnp
from jax import lax
from jax.experimental import pallas as pl
from jax.experimental.pallas import tpu as pltpu
from jax.experimental.pallas import tpu_sc as plsc

N_NODES = 100000
K = 32
D = 128
B = 16384

NC = 2
NS = 16
NW = NC * NS
BPW = B // NW
C = 4
NCH = BPW // C
LANES = 16
NV = D // LANES
INV_K = 1.0 / K
ISLC = 128


def _gcn_body(x_hbm, adj_hbm, table_hbm, out_hbm,
              x_v, edge_v, self_v, nb0, nb1, out0, out1,
              sem_e, sem_s, sem_n0, sem_n1, sem_o0, sem_o1):
    wid = lax.axis_index("s") * NC + lax.axis_index("c")
    base = wid * BPW

    pltpu.sync_copy(x_hbm.at[pl.ds(base, BPW)], x_v)

    for j in range(BPW // ISLC):
        sl = pl.ds(j * ISLC, ISLC)
        pltpu.async_copy(adj_hbm.at[x_v.at[sl]], edge_v.at[sl], sem_e)
    for j in range(BPW // ISLC):
        sl = pl.ds(j * ISLC, ISLC)
        pltpu.async_copy(table_hbm.at[x_v.at[sl]], self_v.at[sl], sem_s)
    for j in range(BPW // ISLC):
        sl = pl.ds(j * ISLC, ISLC)
        pltpu.make_async_copy(adj_hbm.at[x_v.at[sl]], edge_v.at[sl], sem_e).wait()

    def fire_nb(g, nb, sem):
        for q in range(C):
            pltpu.async_copy(table_hbm.at[edge_v.at[g * C + q]], nb.at[q], sem)

    def drain_nb(g, nb, sem):
        for q in range(C):
            pltpu.make_async_copy(
                table_hbm.at[edge_v.at[g * C + q]], nb.at[q], sem).wait()

    def fire_out(g, out_v, sem):
        pltpu.async_copy(out_v, out_hbm.at[pl.ds(base + g * C, C)], sem)

    def drain_out(g, out_v, sem):
        pltpu.make_async_copy(
            out_v, out_hbm.at[pl.ds(base + g * C, C)], sem).wait()

    UNROLL = 4

    def compute(g, nb, out_v):
        for q in range(C):
            def red(k4, accs, q=q):
                new = list(accs)
                for dk in range(UNROLL):
                    row = UNROLL * k4 + dk
                    for d in range(NV):
                        new[d] = new[d] + nb[q, row, pl.ds(d * LANES, LANES)]
                return tuple(new)

            zero = jnp.zeros((LANES,), jnp.float32)
            accs = lax.fori_loop(0, K // UNROLL, red, (zero,) * NV)
            for d in range(NV):
                dsl = pl.ds(d * LANES, LANES)
                out_v[q, dsl] = accs[d] * INV_K + self_v[g * C + q, dsl]

    fire_nb(0, nb0, sem_n0)
    fire_nb(1, nb1, sem_n1)
    for j in range(BPW // ISLC):
        sl = pl.ds(j * ISLC, ISLC)
        pltpu.make_async_copy(table_hbm.at[x_v.at[sl]], self_v.at[sl], sem_s).wait()

    bufs = ((nb0, sem_n0, out0, sem_o0), (nb1, sem_n1, out1, sem_o1))

    def step(i, carry):
        for b, (nb, semn, out_v, semo) in enumerate(bufs):
            g = 2 * i + b

            @pl.when(g >= 2)
            def _():
                drain_out(g - 2, out_v, semo)

            drain_nb(g, nb, semn)
            compute(g, nb, out_v)
            fire_out(g, out_v, semo)

            @pl.when(g + 2 < NCH)
            def _():
                fire_nb(g + 2, nb, semn)

        return carry

    lax.fori_loop(0, NCH // 2, step, 0)
    drain_out(NCH - 2, out0, sem_o0)
    drain_out(NCH - 1, out1, sem_o1)


def kernel(X, adj, table):
    x = jnp.reshape(X, (B,)).astype(jnp.int32)
    adj32 = adj.astype(jnp.int32)
    f = pl.kernel(
        _gcn_body,
        out_type=jax.ShapeDtypeStruct((B, D), jnp.float32),
        mesh=plsc.VectorSubcoreMesh(core_axis_name="c", subcore_axis_name="s"),
        compiler_params=pltpu.CompilerParams(use_tc_tiling_on_sc=False),
        scratch_types=[
            pltpu.VMEM((BPW,), jnp.int32),
            pltpu.VMEM((BPW, K), jnp.int32),
            pltpu.VMEM((BPW, D), jnp.float32),
            pltpu.VMEM((C, K, D), jnp.float32),
            pltpu.VMEM((C, K, D), jnp.float32),
            pltpu.VMEM((C, D), jnp.float32),
            pltpu.VMEM((C, D), jnp.float32),
            pltpu.SemaphoreType.DMA,
            pltpu.SemaphoreType.DMA,
            pltpu.SemaphoreType.DMA,
            pltpu.SemaphoreType.DMA,
            pltpu.SemaphoreType.DMA,
            pltpu.SemaphoreType.DMA,
        ],
    )
    out = f(x, adj32, table)
    return jnp.reshape(out, (B, 1, D))


# speedup vs baseline: 1.8517x; 1.1553x over previous
import jax
import jax.numpy as jnp
from jax import lax
from jax.experimental import pallas as pl
from jax.experimental.pallas import tpu as pltpu
from jax.experimental.pallas import tpu_sc as plsc

N_NODES = 100000
K = 32
D = 128
B = 16384

NC = 2
NS = 16
NW = NC * NS
BPW = B // NW
C = 8
NCH = BPW // C
LANES = 16
NV = D // LANES
INV_K = 1.0 / K
ISLC = 128
UNROLL = 4


def _gcn_body(x_hbm, adj_hbm, table_hbm, out_hbm,
              x_v, edge_v, nb0, nb1, sf0, sf1, out0, out1,
              sem_e, sem_n0, sem_n1, sem_o0, sem_o1):
    wid = lax.axis_index("s") * NC + lax.axis_index("c")
    base = wid * BPW

    pltpu.sync_copy(x_hbm.at[pl.ds(base, BPW)], x_v)

    for j in range(BPW // ISLC):
        sl = pl.ds(j * ISLC, ISLC)
        pltpu.async_copy(adj_hbm.at[x_v.at[sl]], edge_v.at[sl], sem_e)
    for j in range(BPW // ISLC):
        sl = pl.ds(j * ISLC, ISLC)
        pltpu.make_async_copy(adj_hbm.at[x_v.at[sl]], edge_v.at[sl], sem_e).wait()

    def fire_nb(g, nb, sf, sem):
        for q in range(C):
            pltpu.async_copy(table_hbm.at[edge_v.at[g * C + q]], nb.at[q], sem)
        pltpu.async_copy(table_hbm.at[x_v.at[pl.ds(g * C, C)]], sf, sem)

    def drain_nb(g, nb, sf, sem):
        for q in range(C):
            pltpu.make_async_copy(
                table_hbm.at[edge_v.at[g * C + q]], nb.at[q], sem).wait()
        pltpu.make_async_copy(
            table_hbm.at[x_v.at[pl.ds(g * C, C)]], sf, sem).wait()

    def fire_out(g, out_v, sem):
        pltpu.async_copy(out_v, out_hbm.at[pl.ds(base + g * C, C)], sem)

    def drain_out(g, out_v, sem):
        pltpu.make_async_copy(
            out_v, out_hbm.at[pl.ds(base + g * C, C)], sem).wait()

    def compute(g, nb, sf, out_v):
        for q in range(C):
            def red(k4, accs, q=q):
                new = list(accs)
                for dk in range(UNROLL):
                    row = UNROLL * k4 + dk
                    for d in range(NV):
                        new[d] = new[d] + nb[q, row, pl.ds(d * LANES, LANES)]
                return tuple(new)

            zero = jnp.zeros((LANES,), jnp.float32)
            accs = lax.fori_loop(0, K // UNROLL, red, (zero,) * NV)
            for d in range(NV):
                dsl = pl.ds(d * LANES, LANES)
                out_v[q, dsl] = accs[d] * INV_K + sf[q, dsl]

    fire_nb(0, nb0, sf0, sem_n0)
    fire_nb(1, nb1, sf1, sem_n1)

    bufs = ((nb0, sf0, sem_n0, out0, sem_o0), (nb1, sf1, sem_n1, out1, sem_o1))

    def step(i, carry):
        for b, (nb, sf, semn, out_v, semo) in enumerate(bufs):
            g = 2 * i + b

            @pl.when(g >= 2)
            def _():
                drain_out(g - 2, out_v, semo)

            drain_nb(g, nb, sf, semn)
            compute(g, nb, sf, out_v)
            fire_out(g, out_v, semo)

            @pl.when(g + 2 < NCH)
            def _():
                fire_nb(g + 2, nb, sf, semn)

        return carry

    lax.fori_loop(0, NCH // 2, step, 0)
    drain_out(NCH - 2, out0, sem_o0)
    drain_out(NCH - 1, out1, sem_o1)


def kernel(X, adj, table):
    x = jnp.reshape(X, (B,)).astype(jnp.int32)
    adj32 = adj.astype(jnp.int32)
    f = pl.kernel(
        _gcn_body,
        out_type=jax.ShapeDtypeStruct((B, D), jnp.float32),
        mesh=plsc.VectorSubcoreMesh(core_axis_name="c", subcore_axis_name="s"),
        compiler_params=pltpu.CompilerParams(use_tc_tiling_on_sc=False),
        scratch_types=[
            pltpu.VMEM((BPW,), jnp.int32),
            pltpu.VMEM((BPW, K), jnp.int32),
            pltpu.VMEM((C, K, D), jnp.float32),
            pltpu.VMEM((C, K, D), jnp.float32),
            pltpu.VMEM((C, D), jnp.float32),
            pltpu.VMEM((C, D), jnp.float32),
            pltpu.VMEM((C, D), jnp.float32),
            pltpu.VMEM((C, D), jnp.float32),
            pltpu.SemaphoreType.DMA,
            pltpu.SemaphoreType.DMA,
            pltpu.SemaphoreType.DMA,
            pltpu.SemaphoreType.DMA,
            pltpu.SemaphoreType.DMA,
        ],
    )
    out = f(x, adj32, table)
    return jnp.reshape(out, (B, 1, D))


# speedup vs baseline: 2.0529x; 1.1086x over previous
import jax
import jax.numpy as jnp
from jax import lax
from jax.experimental import pallas as pl
from jax.experimental.pallas import tpu as pltpu
from jax.experimental.pallas import tpu_sc as plsc

N_NODES = 100000
K = 32
D = 128
B = 16384

NC = 2
NS = 16
NW = NC * NS
BPW = B // NW
C = 8
NCH = BPW // C
LANES = 16
NV = D // LANES
INV_K = 1.0 / K
ISLC = 128
UNROLL = 4


def _gcn_body(x_hbm, adj_hbm, table_hbm, out_hbm,
              x_v, edge_v, nb0, nb1, nb2, sf0, sf1, sf2, out0, out1, out2,
              sem_e, sem_n0, sem_n1, sem_n2, sem_o0, sem_o1, sem_o2):
    wid = lax.axis_index("s") * NC + lax.axis_index("c")
    base = wid * BPW

    pltpu.sync_copy(x_hbm.at[pl.ds(base, BPW)], x_v)

    for j in range(BPW // ISLC):
        sl = pl.ds(j * ISLC, ISLC)
        pltpu.async_copy(adj_hbm.at[x_v.at[sl]], edge_v.at[sl], sem_e)
    for j in range(BPW // ISLC):
        sl = pl.ds(j * ISLC, ISLC)
        pltpu.make_async_copy(adj_hbm.at[x_v.at[sl]], edge_v.at[sl], sem_e).wait()

    def fire_nb(g, nb, sf, sem):
        for q in range(C):
            pltpu.async_copy(table_hbm.at[edge_v.at[g * C + q]], nb.at[q], sem)
        pltpu.async_copy(table_hbm.at[x_v.at[pl.ds(g * C, C)]], sf, sem)

    def drain_nb(g, nb, sf, sem):
        for q in range(C):
            pltpu.make_async_copy(
                table_hbm.at[edge_v.at[g * C + q]], nb.at[q], sem).wait()
        pltpu.make_async_copy(
            table_hbm.at[x_v.at[pl.ds(g * C, C)]], sf, sem).wait()

    def fire_out(g, out_v, sem):
        pltpu.async_copy(out_v, out_hbm.at[pl.ds(base + g * C, C)], sem)

    def drain_out(g, out_v, sem):
        pltpu.make_async_copy(
            out_v, out_hbm.at[pl.ds(base + g * C, C)], sem).wait()

    def compute(g, nb, sf, out_v):
        def qbody(q, carry):
            def red(k4, accs):
                new = list(accs)
                for dk in range(UNROLL):
                    row = UNROLL * k4 + dk
                    for d in range(NV):
                        new[d] = new[d] + nb[q, row, pl.ds(d * LANES, LANES)]
                return tuple(new)

            zero = jnp.zeros((LANES,), jnp.float32)
            accs = lax.fori_loop(0, K // UNROLL, red, (zero,) * NV)
            for d in range(NV):
                dsl = pl.ds(d * LANES, LANES)
                out_v[q, dsl] = accs[d] * INV_K + sf[q, dsl]
            return carry

        lax.fori_loop(0, C, qbody, 0)

    NB = 3
    fire_nb(0, nb0, sf0, sem_n0)
    fire_nb(1, nb1, sf1, sem_n1)
    fire_nb(2, nb2, sf2, sem_n2)

    bufs = ((nb0, sf0, sem_n0, out0, sem_o0),
            (nb1, sf1, sem_n1, out1, sem_o1),
            (nb2, sf2, sem_n2, out2, sem_o2))

    def step(i, carry):
        for b, (nb, sf, semn, out_v, semo) in enumerate(bufs):
            g = NB * i + b

            @pl.when(g >= NB)
            def _():
                drain_out(g - NB, out_v, semo)

            drain_nb(g, nb, sf, semn)
            compute(g, nb, sf, out_v)
            fire_out(g, out_v, semo)

            @pl.when(g + NB < NCH)
            def _():
                fire_nb(g + NB, nb, sf, semn)

        return carry

    lax.fori_loop(0, NCH // NB, step, 0)
    g_last = (NCH // NB) * NB
    drain_out(g_last - NB, out0, sem_o0)
    drain_nb(g_last, nb0, sf0, sem_n0)
    compute(g_last, nb0, sf0, out0)
    fire_out(g_last, out0, sem_o0)
    drain_out(g_last - 2, out1, sem_o1)
    drain_out(g_last - 1, out2, sem_o2)
    drain_out(g_last, out0, sem_o0)


def kernel(X, adj, table):
    x = jnp.reshape(X, (B,)).astype(jnp.int32)
    adj32 = adj.astype(jnp.int32)
    f = pl.kernel(
        _gcn_body,
        out_type=jax.ShapeDtypeStruct((B, D), jnp.float32),
        mesh=plsc.VectorSubcoreMesh(core_axis_name="c", subcore_axis_name="s"),
        compiler_params=pltpu.CompilerParams(use_tc_tiling_on_sc=False),
        scratch_types=[
            pltpu.VMEM((BPW,), jnp.int32),
            pltpu.VMEM((BPW, K), jnp.int32),
            pltpu.VMEM((C, K, D), jnp.float32),
            pltpu.VMEM((C, K, D), jnp.float32),
            pltpu.VMEM((C, K, D), jnp.float32),
            pltpu.VMEM((C, D), jnp.float32),
            pltpu.VMEM((C, D), jnp.float32),
            pltpu.VMEM((C, D), jnp.float32),
            pltpu.VMEM((C, D), jnp.float32),
            pltpu.VMEM((C, D), jnp.float32),
            pltpu.VMEM((C, D), jnp.float32),
            pltpu.SemaphoreType.DMA,
            pltpu.SemaphoreType.DMA,
            pltpu.SemaphoreType.DMA,
            pltpu.SemaphoreType.DMA,
            pltpu.SemaphoreType.DMA,
            pltpu.SemaphoreType.DMA,
            pltpu.SemaphoreType.DMA,
        ],
    )
    out = f(x, adj32, table)
    return jnp.reshape(out, (B, 1, D))
